# baseline (device time: 311493 ns/iter reference)
import jax
import jax.numpy as jnp
from jax import lax
from jax.experimental import pallas as pl
from jax.experimental.pallas import tpu as pltpu

N_DEV = 4
SQ = 2048
SKV = 2048
D_MODEL = 1024
HQ_TOTAL = 32
HQ_PER = 8
DH = 128
BLK = 64
NBLK = SKV // BLK
NB0, NB1, NB2 = 11, 11, 10
SCALE = 0.08838834764831843

_SEND_PEER_OFFSET = (N_DEV - 1, 2, 1)


def _bdot_qk(q, k):
    return lax.dot_general(
        q, k, (((2,), (2,)), ((0,), (0,))), preferred_element_type=jnp.float32
    )


def _bdot_pv(e, v):
    return lax.dot_general(
        e, v, (((2,), (1,)), ((0,), (0,))), preferred_element_type=jnp.float32
    )


def _dot_qk(q, k):
    return lax.dot_general(
        q, k, (((1,), (1,)), ((), ())), preferred_element_type=jnp.float32
    )


def _dot_pv(e, v):
    return lax.dot_general(
        e, v, (((1,), (0,)), ((), ())), preferred_element_type=jnp.float32
    )


def _mod3_split(a):
    pad = jnp.zeros((1,) + a.shape[1:], a.dtype)
    ar = jnp.concatenate([a, pad]).reshape(NB0, 3, a.shape[1], a.shape[2])
    return ar[:, 0], ar[:, 1], ar[:, 2, :, :][:NB2]


_NAT_BLOCK = [3 * j for j in range(NB0)] + [3 * j + 1 for j in range(NB1)] + [
    3 * j + 2 for j in range(NB2)
]


def _fused_body(
    s_ref,
    x_ref,
    k0_ref,
    ka1_ref,
    ka2_ref,
    v0_ref,
    va1_ref,
    va2_ref,
    w_hbm,
    out_ref,
    w_ref,
    ctx_ref,
    local_sem,
    send_sems,
    recv_sems,
):
    t = pl.program_id(0)
    my = s_ref[0]
    c = t // HQ_PER
    cid = lax.rem(my + c, N_DEV)
    hh = lax.rem(t, HQ_PER)

    def _send(round_idx):
        peer = lax.rem(my + _SEND_PEER_OFFSET[round_idx], N_DEV)
        return pltpu.make_async_remote_copy(
            src_ref=w_hbm,
            dst_ref=w_ref.at[my],
            send_sem=send_sems.at[round_idx],
            recv_sem=recv_sems.at[my],
            device_id=(peer,),
            device_id_type=pl.DeviceIdType.MESH,
        )

    def _recv(offset):
        src = lax.rem(my + offset, N_DEV)
        return pltpu.make_async_remote_copy(
            src_ref=w_hbm,
            dst_ref=w_ref.at[src],
            send_sem=send_sems.at[0],
            recv_sem=recv_sems.at[src],
            device_id=(src,),
            device_id_type=pl.DeviceIdType.MESH,
        )

    @pl.when(t == 0)
    def _():
        cp = pltpu.make_async_copy(w_hbm, w_ref.at[my], local_sem)
        cp.start()
        barrier = pltpu.get_barrier_semaphore()
        for p in range(1, N_DEV):
            peer = lax.rem(my + p, N_DEV)
            pl.semaphore_signal(
                barrier,
                inc=1,
                device_id=(peer,),
                device_id_type=pl.DeviceIdType.MESH,
            )
        pl.semaphore_wait(barrier, N_DEV - 1)
        _send(0).start()
        cp.wait()

    @pl.when(t == HQ_PER)
    def _():
        _recv(1).wait_recv()
        _send(1).start()

    @pl.when(t == 2 * HQ_PER)
    def _():
        _recv(2).wait_recv()
        _send(2).start()

    @pl.when(t == 3 * HQ_PER)
    def _():
        _recv(3).wait_recv()

    x = x_ref[...]
    wq = w_ref[cid, 0, :, pl.ds(hh * DH, DH)]

    q = jnp.dot(x, wq, preferred_element_type=jnp.float32)
    qs = ((q * SCALE).astype(jnp.bfloat16)).reshape(NBLK, BLK, DH)
    qg0, qg1, qg2 = _mod3_split(qs)

    k0, k_a1, k_a2 = k0_ref[0], ka1_ref[0], ka2_ref[0]
    v0, v_a1, v_a2 = v0_ref[0], va1_ref[0], va2_ref[0]
    ks1 = k_a2[BLK:].reshape(NB1, BLK, DH)
    ks2 = k_a1[BLK:].reshape(NB2, BLK, DH)
    vs1 = v_a2[BLK:].reshape(NB1, BLK, DH)
    vs2 = v_a1[BLK:].reshape(NB2, BLK, DH)

    q0 = qg0.reshape(NB0 * BLK, DH)
    e0 = jnp.exp(_dot_qk(q0, k0))
    d0 = jnp.sum(e0, axis=1, keepdims=True)
    c0 = _dot_pv(e0.astype(jnp.bfloat16), v0) / d0

    q1 = qg1.reshape(NB1 * BLK, DH)
    e1 = jnp.exp(_dot_qk(q1, k_a1))
    ed1 = jnp.exp(_bdot_qk(qg1, ks1))
    d1 = jnp.sum(e1, axis=1, keepdims=True) + jnp.sum(ed1, axis=2).reshape(
        NB1 * BLK, 1
    )
    c1 = (
        _dot_pv(e1.astype(jnp.bfloat16), v_a1)
        + _bdot_pv(ed1.astype(jnp.bfloat16), vs1).reshape(NB1 * BLK, DH)
    ) / d1

    q2 = qg2.reshape(NB2 * BLK, DH)
    e2 = jnp.exp(_dot_qk(q2, k_a2))
    ed2 = jnp.exp(_bdot_qk(qg2, ks2))
    d2 = jnp.sum(e2, axis=1, keepdims=True) + jnp.sum(ed2, axis=2).reshape(
        NB2 * BLK, 1
    )
    c2 = (
        _dot_pv(e2.astype(jnp.bfloat16), v_a2)
        + _bdot_pv(ed2.astype(jnp.bfloat16), vs2).reshape(NB2 * BLK, DH)
    ) / d2

    ctx = jnp.concatenate([c0, c1, c2], axis=0).astype(jnp.bfloat16)
    ah = cid * HQ_PER + hh
    ctx_ref[:, pl.ds(ah * DH, DH)] = ctx

    @pl.when(t == HQ_TOTAL - 1)
    def _():
        acc = jnp.dot(
            ctx_ref[:, 0:D_MODEL],
            w_ref[0, 1],
            preferred_element_type=jnp.float32,
        )
        for j in range(1, N_DEV):
            acc += jnp.dot(
                ctx_ref[:, j * D_MODEL : (j + 1) * D_MODEL],
                w_ref[j, 1],
                preferred_element_type=jnp.float32,
            )
        for j, nat in enumerate(_NAT_BLOCK):
            out_ref[nat * BLK : (nat + 1) * BLK, :] = acc[
                j * BLK : (j + 1) * BLK, :
            ]
        for r in range(N_DEV - 1):
            _send(r).wait_send()


def _group_kv(a):
    ab = (
        a.astype(jnp.bfloat16)
        .reshape(NBLK, BLK, HQ_TOTAL, DH)
        .transpose(2, 0, 1, 3)
    )
    s0 = ab[:, 0::3]
    s1 = ab[:, 1::3]
    s2 = ab[:, 2::3]
    a0 = s0.reshape(HQ_TOTAL, NB0 * BLK, DH)
    a1 = jnp.concatenate([ab[:, :1], s2], axis=1).reshape(
        HQ_TOTAL, (1 + NB2) * BLK, DH
    )
    a2 = jnp.concatenate([ab[:, :1], s1], axis=1).reshape(
        HQ_TOTAL, (1 + NB1) * BLK, DH
    )
    return a0, a1, a2


def kernel(x, Wq, K_ext, V_ext, Wo):
    my = lax.axis_index("i")
    xb = x[0].astype(jnp.bfloat16)
    k0, ka1, ka2 = _group_kv(lax.dynamic_index_in_dim(K_ext, my, 0, keepdims=False))
    v0, va1, va2 = _group_kv(lax.dynamic_index_in_dim(V_ext, my, 0, keepdims=False))
    w = jnp.stack([Wq.astype(jnp.bfloat16), Wo.astype(jnp.bfloat16)])
    my_arr = jnp.reshape(my, (1,)).astype(jnp.int32)

    def _kv_map(t, s):
        return (lax.rem(s[0] * HQ_PER + t, HQ_TOTAL), 0, 0)

    kv_specs = [
        pl.BlockSpec((1, M, DH), _kv_map)
        for M in (NB0 * BLK, (1 + NB2) * BLK, (1 + NB1) * BLK)
    ]
    grid_spec = pltpu.PrefetchScalarGridSpec(
        num_scalar_prefetch=1,
        grid=(HQ_TOTAL,),
        in_specs=[
            pl.BlockSpec((SQ, D_MODEL), lambda t, s: (0, 0)),
            *kv_specs,
            *kv_specs,
            pl.BlockSpec(memory_space=pltpu.MemorySpace.HBM),
        ],
        out_specs=pl.BlockSpec((SQ, D_MODEL), lambda t, s: (0, 0)),
        scratch_shapes=[
            pltpu.VMEM((N_DEV, 2, D_MODEL, D_MODEL), jnp.bfloat16),
            pltpu.VMEM((SQ, HQ_TOTAL * DH), jnp.bfloat16),
            pltpu.SemaphoreType.DMA,
            pltpu.SemaphoreType.DMA((N_DEV - 1,)),
            pltpu.SemaphoreType.DMA((N_DEV,)),
        ],
    )
    out = pl.pallas_call(
        _fused_body,
        grid_spec=grid_spec,
        out_shape=jax.ShapeDtypeStruct((SQ, D_MODEL), jnp.float32),
        compiler_params=pltpu.CompilerParams(
            dimension_semantics=("arbitrary",),
            vmem_limit_bytes=63 * 1024 * 1024,
            collective_id=0,
        ),
    )(my_arr, xb, k0, ka1, ka2, v0, va1, va2, w)

    return out[None]


# device time: 289338 ns/iter; 1.0766x vs baseline; 1.0766x over previous
import jax
import jax.numpy as jnp
from jax import lax
from jax.experimental import pallas as pl
from jax.experimental.pallas import tpu as pltpu

N_DEV = 4
SQ = 2048
SKV = 2048
D_MODEL = 1024
HQ_TOTAL = 32
HQ_PER = 8
DH = 128
BLK = 64
NBLK = SKV // BLK
NB0, NB1, NB2 = 11, 11, 10
SCALE = 0.08838834764831843

_SEND_PEER_OFFSET = (N_DEV - 1, 2, 1)


def _bdot_qk(q, k):
    return lax.dot_general(
        q, k, (((2,), (2,)), ((0,), (0,))), preferred_element_type=jnp.float32
    )


def _bdot_pv(e, v):
    return lax.dot_general(
        e, v, (((2,), (1,)), ((0,), (0,))), preferred_element_type=jnp.float32
    )


def _dot_qk(q, k):
    return lax.dot_general(
        q, k, (((1,), (1,)), ((), ())), preferred_element_type=jnp.float32
    )


def _dot_pv(e, v):
    return lax.dot_general(
        e, v, (((1,), (0,)), ((), ())), preferred_element_type=jnp.float32
    )


def _mod3_split(a):
    pad = jnp.zeros((1,) + a.shape[1:], a.dtype)
    ar = jnp.concatenate([a, pad]).reshape(NB0, 3, a.shape[1], a.shape[2])
    return ar[:, 0], ar[:, 1], ar[:, 2, :, :][:NB2]


_NAT_BLOCK = [3 * j for j in range(NB0)] + [3 * j + 1 for j in range(NB1)] + [
    3 * j + 2 for j in range(NB2)
]


def _fused_body(
    s_ref,
    x_ref,
    k_ref,
    v_ref,
    w_hbm,
    out_ref,
    w_ref,
    ctx_ref,
    local_sem,
    send_sems,
    recv_sems,
):
    t = pl.program_id(0)
    my = s_ref[0]
    c = t // HQ_PER
    cid = lax.rem(my + c, N_DEV)
    hh = lax.rem(t, HQ_PER)

    def _send(round_idx):
        peer = lax.rem(my + _SEND_PEER_OFFSET[round_idx], N_DEV)
        return pltpu.make_async_remote_copy(
            src_ref=w_hbm,
            dst_ref=w_ref.at[my],
            send_sem=send_sems.at[round_idx],
            recv_sem=recv_sems.at[my],
            device_id=(peer,),
            device_id_type=pl.DeviceIdType.MESH,
        )

    def _recv(offset):
        src = lax.rem(my + offset, N_DEV)
        return pltpu.make_async_remote_copy(
            src_ref=w_hbm,
            dst_ref=w_ref.at[src],
            send_sem=send_sems.at[0],
            recv_sem=recv_sems.at[src],
            device_id=(src,),
            device_id_type=pl.DeviceIdType.MESH,
        )

    @pl.when(t == 0)
    def _():
        cp = pltpu.make_async_copy(w_hbm, w_ref.at[my], local_sem)
        cp.start()
        barrier = pltpu.get_barrier_semaphore()
        for p in range(1, N_DEV):
            peer = lax.rem(my + p, N_DEV)
            pl.semaphore_signal(
                barrier,
                inc=1,
                device_id=(peer,),
                device_id_type=pl.DeviceIdType.MESH,
            )
        pl.semaphore_wait(barrier, N_DEV - 1)
        _send(0).start()
        cp.wait()

    @pl.when(t == HQ_PER)
    def _():
        _recv(1).wait_recv()
        _send(1).start()

    @pl.when(t == 2 * HQ_PER)
    def _():
        _recv(2).wait_recv()
        _send(2).start()

    @pl.when(t == 3 * HQ_PER)
    def _():
        _recv(3).wait_recv()

    x = x_ref[...]
    wq = w_ref[cid, 0, :, pl.ds(hh * DH, DH)]
    kblk = k_ref[0].reshape(NBLK, BLK, DH)
    vblk = v_ref[0].reshape(NBLK, BLK, DH)

    q = jnp.dot(x, wq, preferred_element_type=jnp.float32)
    qs = ((q * SCALE).astype(jnp.bfloat16)).reshape(NBLK, BLK, DH)

    ks0, ks1, ks2 = _mod3_split(kblk)
    vs0, vs1, vs2 = _mod3_split(vblk)
    qg0, qg1, qg2 = _mod3_split(qs)

    k0 = ks0.reshape(NB0 * BLK, DH)
    k_a1 = jnp.concatenate([kblk[0:1], ks2]).reshape((1 + NB2) * BLK, DH)
    k_a2 = jnp.concatenate([kblk[0:1], ks1]).reshape((1 + NB1) * BLK, DH)
    v0 = vs0.reshape(NB0 * BLK, DH)
    v_a1 = jnp.concatenate([vblk[0:1], vs2]).reshape((1 + NB2) * BLK, DH)
    v_a2 = jnp.concatenate([vblk[0:1], vs1]).reshape((1 + NB1) * BLK, DH)

    q0 = qg0.reshape(NB0 * BLK, DH)
    e0 = jnp.exp(_dot_qk(q0, k0))
    d0 = jnp.sum(e0, axis=1, keepdims=True)
    c0 = _dot_pv(e0.astype(jnp.bfloat16), v0) / d0

    q1 = qg1.reshape(NB1 * BLK, DH)
    e1 = jnp.exp(_dot_qk(q1, k_a1))
    ed1 = jnp.exp(_bdot_qk(qg1, ks1))
    d1 = jnp.sum(e1, axis=1, keepdims=True) + jnp.sum(ed1, axis=2).reshape(
        NB1 * BLK, 1
    )
    c1 = (
        _dot_pv(e1.astype(jnp.bfloat16), v_a1)
        + _bdot_pv(ed1.astype(jnp.bfloat16), vs1).reshape(NB1 * BLK, DH)
    ) / d1

    q2 = qg2.reshape(NB2 * BLK, DH)
    e2 = jnp.exp(_dot_qk(q2, k_a2))
    ed2 = jnp.exp(_bdot_qk(qg2, ks2))
    d2 = jnp.sum(e2, axis=1, keepdims=True) + jnp.sum(ed2, axis=2).reshape(
        NB2 * BLK, 1
    )
    c2 = (
        _dot_pv(e2.astype(jnp.bfloat16), v_a2)
        + _bdot_pv(ed2.astype(jnp.bfloat16), vs2).reshape(NB2 * BLK, DH)
    ) / d2

    ctx = jnp.concatenate([c0, c1, c2], axis=0).astype(jnp.bfloat16)
    ah = cid * HQ_PER + hh
    ctx_ref[:, pl.ds(ah * DH, DH)] = ctx

    @pl.when(t == HQ_TOTAL - 1)
    def _():
        acc = jnp.dot(
            ctx_ref[:, 0:D_MODEL],
            w_ref[0, 1],
            preferred_element_type=jnp.float32,
        )
        for j in range(1, N_DEV):
            acc += jnp.dot(
                ctx_ref[:, j * D_MODEL : (j + 1) * D_MODEL],
                w_ref[j, 1],
                preferred_element_type=jnp.float32,
            )
        for j, nat in enumerate(_NAT_BLOCK):
            out_ref[nat * BLK : (nat + 1) * BLK, :] = acc[
                j * BLK : (j + 1) * BLK, :
            ]
        for r in range(N_DEV - 1):
            _send(r).wait_send()


def kernel(x, Wq, K_ext, V_ext, Wo):
    my = lax.axis_index("i")
    xb = x[0].astype(jnp.bfloat16)
    kt = (
        lax.dynamic_index_in_dim(K_ext, my, 0, keepdims=False)
        .astype(jnp.bfloat16)
        .transpose(1, 0, 2)
    )
    vt = (
        lax.dynamic_index_in_dim(V_ext, my, 0, keepdims=False)
        .astype(jnp.bfloat16)
        .transpose(1, 0, 2)
    )
    w = jnp.stack([Wq.astype(jnp.bfloat16), Wo.astype(jnp.bfloat16)])
    my_arr = jnp.reshape(my, (1,)).astype(jnp.int32)

    def _kv_map(t, s):
        return (lax.rem(s[0] * HQ_PER + t, HQ_TOTAL), 0, 0)

    grid_spec = pltpu.PrefetchScalarGridSpec(
        num_scalar_prefetch=1,
        grid=(HQ_TOTAL,),
        in_specs=[
            pl.BlockSpec((SQ, D_MODEL), lambda t, s: (0, 0)),
            pl.BlockSpec((1, SKV, DH), _kv_map),
            pl.BlockSpec((1, SKV, DH), _kv_map),
            pl.BlockSpec(memory_space=pltpu.MemorySpace.HBM),
        ],
        out_specs=pl.BlockSpec((SQ, D_MODEL), lambda t, s: (0, 0)),
        scratch_shapes=[
            pltpu.VMEM((N_DEV, 2, D_MODEL, D_MODEL), jnp.bfloat16),
            pltpu.VMEM((SQ, HQ_TOTAL * DH), jnp.bfloat16),
            pltpu.SemaphoreType.DMA,
            pltpu.SemaphoreType.DMA((N_DEV - 1,)),
            pltpu.SemaphoreType.DMA((N_DEV,)),
        ],
    )
    out = pl.pallas_call(
        _fused_body,
        grid_spec=grid_spec,
        out_shape=jax.ShapeDtypeStruct((SQ, D_MODEL), jnp.float32),
        compiler_params=pltpu.CompilerParams(
            dimension_semantics=("arbitrary",),
            vmem_limit_bytes=63 * 1024 * 1024,
            collective_id=0,
        ),
    )(my_arr, xb, kt, vt, w)

    return out[None]
